# Initial kernel scaffold; baseline (speedup 1.0000x reference)
import jax
import jax.numpy as jnp
from jax.experimental import pallas as pl


def kernel(token_ids, weight):
    raise NotImplementedError("write your pallas kernel here")


# speedup vs baseline: 1.0787x; 1.0787x over previous
import functools

import jax
import jax.numpy as jnp
from jax import lax
from jax.experimental import pallas as pl
from jax.experimental.pallas import tpu as pltpu
from jax.experimental.pallas import tpu_sc as plsc

_INFO = plsc.get_sparse_core_info()
_NC = _INFO.num_cores
_NS = _INFO.num_subcores
_NW = _NC * _NS

_D = 32
_CHUNK = 128


def _make_gather(num_rows: int, vocab: int):
    b_per_w = num_rows // _NW
    n_chunks = b_per_w // _CHUNK
    mesh = plsc.VectorSubcoreMesh(core_axis_name="c", subcore_axis_name="s")

    @functools.partial(
        pl.kernel,
        mesh=mesh,
        out_type=jax.ShapeDtypeStruct((num_rows, _D), jnp.float32),
        scratch_types=[
            pltpu.VMEM((n_chunks, _CHUNK), jnp.int32),
            pltpu.VMEM((_CHUNK, _D), jnp.float32),
            pltpu.VMEM((_CHUNK, _D), jnp.float32),
            pltpu.SemaphoreType.DMA,
            pltpu.SemaphoreType.DMA,
        ],
        compiler_params=pltpu.CompilerParams(use_tc_tiling_on_sc=False),
    )
    def emb(table_hbm, idx_hbm, out_hbm, idx_v, rows0, rows1, sem0, sem1):
        wid = lax.axis_index("s") * _NC + lax.axis_index("c")
        base = wid * b_per_w
        pltpu.sync_copy(idx_hbm.at[wid], idx_v)

        bufs = (rows0, rows1)
        sems = (sem0, sem1)

        pltpu.async_copy(table_hbm.at[idx_v.at[0]], rows0, sem0)
        pltpu.async_copy(table_hbm.at[idx_v.at[1]], rows1, sem1)

        def step(g, _):
            for b in range(2):
                j = g * 2 + b
                buf, sem = bufs[b], sems[b]
                pltpu.make_async_copy(table_hbm.at[idx_v.at[j]], buf, sem).wait()
                pltpu.sync_copy(buf, out_hbm.at[pl.ds(base + j * _CHUNK, _CHUNK)])

                @pl.when(j + 2 < n_chunks)
                def _():
                    pltpu.async_copy(table_hbm.at[idx_v.at[j + 2]], buf, sem)

            return 0

        lax.fori_loop(0, n_chunks // 2, step, 0)

    return emb


def kernel(token_ids, weight):
    shape = token_ids.shape
    dim = weight.shape[1]
    ids = token_ids.reshape(-1).astype(jnp.int32)
    num_rows = ids.shape[0]
    idx3 = ids.reshape(_NW, num_rows // _NW // _CHUNK, _CHUNK)
    out = _make_gather(num_rows, weight.shape[0])(weight, idx3)
    return out.reshape(shape + (dim,))


# speedup vs baseline: 1.1134x; 1.0322x over previous
import functools

import jax
import jax.numpy as jnp
from jax import lax
from jax.experimental import pallas as pl
from jax.experimental.pallas import tpu as pltpu
from jax.experimental.pallas import tpu_sc as plsc

_INFO = plsc.get_sparse_core_info()
_NC = _INFO.num_cores
_NS = _INFO.num_subcores
_NW = _NC * _NS

_D = 32
_CHUNK = 256
_NBUF = 4


def _make_gather(num_rows: int):
    b_per_w = num_rows // _NW
    n_chunks = b_per_w // _CHUNK
    assert n_chunks % _NBUF == 0
    mesh = plsc.VectorSubcoreMesh(core_axis_name="c", subcore_axis_name="s")

    row_bufs = [pltpu.VMEM((_CHUNK, _D), jnp.float32) for _ in range(_NBUF)]
    gather_sems = [pltpu.SemaphoreType.DMA for _ in range(_NBUF)]
    store_sems = [pltpu.SemaphoreType.DMA for _ in range(_NBUF)]

    @functools.partial(
        pl.kernel,
        mesh=mesh,
        out_type=jax.ShapeDtypeStruct((num_rows, _D), jnp.float32),
        scratch_types=[pltpu.VMEM((n_chunks, _CHUNK), jnp.int32)]
        + row_bufs + gather_sems + store_sems,
        compiler_params=pltpu.CompilerParams(use_tc_tiling_on_sc=False),
    )
    def emb(table_hbm, idx_hbm, out_hbm, idx_v, *refs):
        bufs = refs[:_NBUF]
        gsems = refs[_NBUF:2 * _NBUF]
        ssems = refs[2 * _NBUF:]
        wid = lax.axis_index("s") * _NC + lax.axis_index("c")
        base = wid * b_per_w
        pltpu.sync_copy(idx_hbm.at[wid], idx_v)

        for b in range(_NBUF):
            pltpu.async_copy(table_hbm.at[idx_v.at[b]], bufs[b], gsems[b])

        def step(g, _):
            for b in range(_NBUF):
                j = g * _NBUF + b
                buf, gsem, ssem = bufs[b], gsems[b], ssems[b]
                pltpu.make_async_copy(table_hbm.at[idx_v.at[j]], buf, gsem).wait()
                dst = out_hbm.at[pl.ds(base + j * _CHUNK, _CHUNK)]
                pltpu.async_copy(buf, dst, ssem)

                @pl.when(j + _NBUF < n_chunks)
                def _():
                    pltpu.make_async_copy(buf, dst, ssem).wait()
                    pltpu.async_copy(
                        table_hbm.at[idx_v.at[j + _NBUF]], buf, gsem)

            return 0

        lax.fori_loop(0, n_chunks // _NBUF, step, 0)

        for b in range(_NBUF):
            j = n_chunks - _NBUF + b
            dst = out_hbm.at[pl.ds(base + j * _CHUNK, _CHUNK)]
            pltpu.make_async_copy(bufs[b], dst, ssems[b]).wait()

    return emb


def kernel(token_ids, weight):
    shape = token_ids.shape
    dim = weight.shape[1]
    ids = token_ids.reshape(-1).astype(jnp.int32)
    num_rows = ids.shape[0]
    idx3 = ids.reshape(_NW, num_rows // _NW // _CHUNK, _CHUNK)
    out = _make_gather(num_rows)(weight, idx3)
    return out.reshape(shape + (dim,))


# speedup vs baseline: 1.8392x; 1.6518x over previous
import functools

import jax
import jax.numpy as jnp
from jax import lax
from jax.experimental import pallas as pl
from jax.experimental.pallas import tpu as pltpu
from jax.experimental.pallas import tpu_sc as plsc

_INFO = plsc.get_sparse_core_info()
_NC = _INFO.num_cores
_NS = _INFO.num_subcores
_NW = _NC * _NS

_S = 50
_T = 16384
_D = 32
_TL = 128
_DS = 8
_NDT = _D // _DS
_NTT = _T // _TL
_NBLK = _S * _NTT
_BPW = _NBLK // _NW


def _make_gather():
    mesh = plsc.VectorSubcoreMesh(core_axis_name="c", subcore_axis_name="s")

    @functools.partial(
        pl.kernel,
        mesh=mesh,
        out_type=jax.ShapeDtypeStruct((_S, _NDT, _NTT, _DS * _TL),
                                      jnp.float32),
        scratch_types=[
            pltpu.VMEM((_BPW, _TL), jnp.int32),
            pltpu.VMEM((_TL, _D), jnp.float32),
            pltpu.VMEM((_TL, _D), jnp.float32),
            pltpu.VMEM((_D * _TL,), jnp.float32),
            pltpu.VMEM((_D * _TL,), jnp.float32),
            pltpu.SemaphoreType.DMA,
            pltpu.SemaphoreType.DMA,
            pltpu.SemaphoreType.DMA,
            pltpu.SemaphoreType.DMA,
        ],
        compiler_params=pltpu.CompilerParams(use_tc_tiling_on_sc=False,
                                             needs_layout_passes=False),
    )
    def emb(table_hbm, idx_hbm, out_hbm, idx_v, rows0, rows1, tb0, tb1,
            gsem0, gsem1, ssem0, ssem1):
        wid = lax.axis_index("s") * _NC + lax.axis_index("c")
        base = wid * _BPW
        pltpu.sync_copy(idx_hbm.at[wid], idx_v)

        rows = (rows0, rows1)
        tbs = (tb0, tb1)
        gsems = (gsem0, gsem1)
        ssems = (ssem0, ssem1)

        base_lo = lax.iota(jnp.int32, 16) * _TL
        base_hi = base_lo + 16 * _TL

        pltpu.async_copy(table_hbm.at[idx_v.at[0]], rows0, gsem0)
        pltpu.async_copy(table_hbm.at[idx_v.at[1]], rows1, gsem1)

        def step(k2, _):
            for b in range(2):
                k = k2 * 2 + b
                rbuf, tbuf, gsem, ssem = rows[b], tbs[b], gsems[b], ssems[b]
                pltpu.make_async_copy(table_hbm.at[idx_v.at[k]], rbuf,
                                      gsem).wait()

                @pl.when(k >= 2)
                def _():
                    bid2 = base + k - 2
                    s2 = bid2 // _NTT
                    tt2 = lax.rem(bid2, _NTT)
                    for dt in range(_NDT):
                        pltpu.make_async_copy(
                            tbuf.at[pl.ds(dt * _DS * _TL, _DS * _TL)],
                            out_hbm.at[s2, dt, tt2], ssem).wait()

                def tr(tj, _):
                    for u in range(8):
                        t = tj * 8 + u
                        plsc.store_scatter(tbuf, [base_lo + t],
                                           rbuf[t, pl.ds(0, 16)])
                        plsc.store_scatter(tbuf, [base_hi + t],
                                           rbuf[t, pl.ds(16, 16)])
                    return 0

                lax.fori_loop(0, _TL // 8, tr, 0)

                bid = base + k
                s = bid // _NTT
                tt = lax.rem(bid, _NTT)
                for dt in range(_NDT):
                    pltpu.async_copy(
                        tbuf.at[pl.ds(dt * _DS * _TL, _DS * _TL)],
                        out_hbm.at[s, dt, tt], ssem)

                @pl.when(k + 2 < _BPW)
                def _():
                    pltpu.async_copy(table_hbm.at[idx_v.at[k + 2]], rbuf, gsem)

            return 0

        lax.fori_loop(0, _BPW // 2, step, 0)

        for b in range(2):
            k = _BPW - 2 + b
            bid = base + k
            s = bid // _NTT
            tt = lax.rem(bid, _NTT)
            for dt in range(_NDT):
                pltpu.make_async_copy(
                    tbs[b].at[pl.ds(dt * _DS * _TL, _DS * _TL)],
                    out_hbm.at[s, dt, tt], ssems[b]).wait()

    return emb


def kernel(token_ids, weight):
    idsw = token_ids.astype(jnp.int32).T.reshape(_NW, _BPW, _TL)
    out5 = _make_gather()(weight, idsw)
    out5 = out5.reshape(_S, _NDT, _NTT, _DS, _TL)
    return out5.transpose(2, 4, 0, 1, 3).reshape(_T, _S, _D)
